# baseline (speedup 1.0000x reference)
import functools

import jax
import jax.numpy as jnp
from jax import lax
from jax.experimental import pallas as pl
from jax.experimental.pallas import tpu as pltpu
from jax.experimental.pallas import tpu_sc as plsc

N_CELL = 50000
N_NET = 10000
E_PIN = 800000
E_PT = 200000

NC, NS, L = 2, 16, 16
NW = NC * NS

E_PAD = 802816
ET = E_PAD // NW
KC = 1792
NGRP = KC // 128
NCHP = ET // KC
CAP = ET
NB = NW + 1
CPAD = 48
K2 = 256
R_CELL, R_NET = 1600, 320
PAD_SRC, PAD_DST = 32 * R_CELL, 32 * R_NET

EP_PAD = 204800

_SC_MESH = functools.partial(
    plsc.VectorSubcoreMesh, core_axis_name="c", subcore_axis_name="s")
_UNTILE = pltpu.CompilerParams(use_tc_tiling_on_sc=False)


def _wid():
    return lax.axis_index("s") * NC + lax.axis_index("c")


def _bucket(own, R):
    q = (own.astype(jnp.float32) * (1.0 / R)).astype(jnp.int32)
    q = jnp.where(own >= (q + 1) * R, q + 1, q)
    q = jnp.where(own < q * R, q - 1, q)
    return q


def _make_degree_kernel():
    ZC, ZC_LAST = 3128, N_CELL - 15 * 3128
    ZN, ZN_LAST = 624, N_NET - 15 * 624

    def body(src_hbm, dst_hbm, dc_hbm, dn_hbm,
             sb, db, vb, i2, zb, acc_c, acc_n, sem):
        cid = lax.axis_index("c")
        sid = lax.axis_index("s")
        ebase = _wid() * ET

        def zv(i, _):
            zb[pl.ds(i * L, L)] = jnp.zeros((L,), jnp.float32)
            return 0

        lax.fori_loop(0, 3136 // L, zv, 0)

        @pl.when(sid < NS - 1)
        def _():
            pltpu.sync_copy(zb.at[pl.ds(0, ZC)], acc_c.at[pl.ds(sid * ZC, ZC)])
            pltpu.sync_copy(zb.at[pl.ds(0, ZN)], acc_n.at[pl.ds(sid * ZN, ZN)])

        @pl.when(sid == NS - 1)
        def _():
            pltpu.sync_copy(zb.at[pl.ds(0, ZC_LAST)],
                            acc_c.at[pl.ds(sid * ZC, ZC_LAST)])
            pltpu.sync_copy(zb.at[pl.ds(0, ZN_LAST)],
                            acc_n.at[pl.ds(sid * ZN, ZN_LAST)])

        plsc.subcore_barrier()

        def chunk(ch, _):
            base = ebase + ch * KC
            for idx_hbm, acc, nmax in ((src_hbm, acc_c, N_CELL),
                                       (dst_hbm, acc_n, N_NET)):
                pltpu.sync_copy(idx_hbm.at[pl.ds(base, KC)], sb)

                def bv(i, _):
                    v = sb[pl.ds(i * L, L)]
                    vb[pl.ds(i * L, L)] = jnp.where(v < nmax, 1.0, 0.0)
                    g = i * L - (i // 8) * 128
                    i2[i // 8, pl.ds(g, L)] = jnp.minimum(v, nmax - 1)
                    return 0

                lax.fori_loop(0, KC // L, bv, 0)

                def sc(g, _):
                    pltpu.sync_copy(vb.at[pl.ds(g * 128, 128)],
                                    acc.at[i2.at[g]], add=True)
                    return 0

                lax.fori_loop(0, NGRP, sc, 0)
            return 0

        lax.fori_loop(0, NCHP, chunk, 0)
        plsc.subcore_barrier()

        @pl.when(sid < NS - 1)
        def _():
            pltpu.sync_copy(acc_c.at[pl.ds(sid * ZC, ZC)], zb.at[pl.ds(0, ZC)])
            pltpu.sync_copy(zb.at[pl.ds(0, ZC)],
                            dc_hbm.at[pl.ds(cid * N_CELL + sid * ZC, ZC)])
            pltpu.sync_copy(acc_n.at[pl.ds(sid * ZN, ZN)], zb.at[pl.ds(0, ZN)])
            pltpu.sync_copy(zb.at[pl.ds(0, ZN)],
                            dn_hbm.at[pl.ds(cid * N_NET + sid * ZN, ZN)])

        @pl.when(sid == NS - 1)
        def _():
            pltpu.sync_copy(acc_c.at[pl.ds(sid * ZC, ZC_LAST)],
                            zb.at[pl.ds(0, ZC_LAST)])
            pltpu.sync_copy(zb.at[pl.ds(0, ZC_LAST)],
                            dc_hbm.at[pl.ds(cid * N_CELL + sid * ZC, ZC_LAST)])
            pltpu.sync_copy(acc_n.at[pl.ds(sid * ZN, ZN_LAST)],
                            zb.at[pl.ds(0, ZN_LAST)])
            pltpu.sync_copy(zb.at[pl.ds(0, ZN_LAST)],
                            dn_hbm.at[pl.ds(cid * N_NET + sid * ZN, ZN_LAST)])

    return pl.kernel(
        body,
        out_type=(jax.ShapeDtypeStruct((NC * N_CELL,), jnp.float32),
                  jax.ShapeDtypeStruct((NC * N_NET,), jnp.float32)),
        mesh=_SC_MESH(),
        scratch_types=[
            pltpu.VMEM((KC,), jnp.int32),
            pltpu.VMEM((KC,), jnp.int32),
            pltpu.VMEM((KC,), jnp.float32),
            pltpu.VMEM((NGRP, 128), jnp.int32),
            pltpu.VMEM((3136,), jnp.float32),
            pltpu.VMEM_SHARED((N_CELL,), jnp.float32),
            pltpu.VMEM_SHARED((N_NET,), jnp.float32),
            pltpu.SemaphoreType.DMA,
        ],
    )


def _make_partition_kernel():
    def body(src_hbm, dst_hbm, pkc_hbm, cntc_hbm, pkn_hbm, cntn_hbm,
             sb, db, bbC, bbN, pkbC, pkbN, tg, cb, cC, cN, sem):
        wid = _wid()
        ebase = wid * ET

        def ci(i, _):
            cC[i] = 0
            cN[i] = 0
            return 0

        lax.fori_loop(0, NB, ci, 0)

        def chunk(ch, _):
            base = ebase + ch * KC
            pltpu.sync_copy(src_hbm.at[pl.ds(base, KC)], sb)
            pltpu.sync_copy(dst_hbm.at[pl.ds(base, KC)], db)

            def bv(i, _):
                sv = sb[pl.ds(i * L, L)]
                dv = db[pl.ds(i * L, L)]
                bC = _bucket(sv, R_CELL)
                bN = _bucket(dv, R_NET)
                bbC[pl.ds(i * L, L)] = bC
                bbN[pl.ds(i * L, L)] = bN
                pkbC[pl.ds(i * L, L)] = ((sv - bC * R_CELL) << 16) | dv
                pkbN[pl.ds(i * L, L)] = ((dv - bN * R_NET) << 16) | sv
                return 0

            lax.fori_loop(0, KC // L, bv, 0)

            for bb, pkb, csm, pk_hbm in ((bbC, pkbC, cC, pkc_hbm),
                                         (bbN, pkbN, cN, pkn_hbm)):
                def grp(g, _):
                    for v8 in range(8):
                        k0 = g * 128 + v8 * 16
                        bv16 = bb[pl.ds(k0, L)]
                        tv = jnp.zeros((L,), jnp.int32)
                        lanes = lax.iota(jnp.int32, L)
                        for j in range(L):
                            b = bv16[j]
                            c = csm[b]
                            csm[b] = c + 1
                            tgt = (wid * NB + b) * CAP + c
                            tv = jnp.where(lanes == j, tgt, tv)
                        tg[g, pl.ds(v8 * 16, L)] = tv

                    pltpu.sync_copy(pkb.at[pl.ds(g * 128, 128)],
                                    pk_hbm.at[tg.at[g]])
                    return 0

                lax.fori_loop(0, NGRP, grp, 0)
            return 0

        lax.fori_loop(0, NCHP, chunk, 0)

        for csm, cnt_hbm in ((cC, cntc_hbm), (cN, cntn_hbm)):
            lanes = lax.iota(jnp.int32, L)
            for v in range(CPAD // L):
                tv = jnp.zeros((L,), jnp.int32)
                for j in range(L):
                    idx = v * L + j
                    if idx < NB:
                        tv = jnp.where(lanes == j, csm[idx], tv)
                cb[pl.ds(v * L, L)] = tv
            pltpu.sync_copy(cb, cnt_hbm.at[pl.ds(wid * CPAD, CPAD)])

    return pl.kernel(
        body,
        out_type=(
            jax.ShapeDtypeStruct((NW * NB * CAP,), jnp.int32),
            jax.ShapeDtypeStruct((NW * CPAD,), jnp.int32),
            jax.ShapeDtypeStruct((NW * NB * CAP,), jnp.int32),
            jax.ShapeDtypeStruct((NW * CPAD,), jnp.int32),
        ),
        mesh=_SC_MESH(),
        scratch_types=[
            pltpu.VMEM((KC,), jnp.int32),
            pltpu.VMEM((KC,), jnp.int32),
            pltpu.VMEM((KC,), jnp.int32),
            pltpu.VMEM((KC,), jnp.int32),
            pltpu.VMEM((KC,), jnp.int32),
            pltpu.VMEM((KC,), jnp.int32),
            pltpu.VMEM((NGRP, 128), jnp.int32),
            pltpu.VMEM((CPAD,), jnp.int32),
            pltpu.SMEM((NB,), jnp.int32),
            pltpu.SMEM((NB,), jnp.int32),
            pltpu.SemaphoreType.DMA,
        ],
    )


def _make_stats_kernel(N_own, N_tab, R, C, P):
    R_last = N_own - (NW - 1) * R
    assert R % 8 == 0 and R_last % 8 == 0

    def body(pk_hbm, cnt_hbm, *rest):
        tabs = rest[:P]
        (sum_hbm, max_hbm, cb, pkb, ib2, rows, acc_s, acc_m, sem) = rest[P:]
        b = _wid()
        lo = b * R

        pltpu.sync_copy(cnt_hbm, cb)

        for p in range(P):
            tab = tabs[p]

            def zvec(i, _):
                z = jnp.zeros((L,), jnp.float32)
                for cc in range(C // L):
                    acc_s[i, pl.ds(cc * L, L)] = z
                    acc_m[i, pl.ds(cc * L, L)] = z - 3.0e38
                return 0

            lax.fori_loop(0, R, zvec, 0)

            def region(t, _):
                n = cb[pl.ds(t * CPAD + b, L)][0]

                def asub(j, _):
                    off = (t * NB + b) * CAP + j * K2
                    pltpu.sync_copy(pk_hbm.at[pl.ds(off, K2)],
                                    pkb.at[pl.ds(0, K2)])

                    def ivec(i, _):
                        pk = pkb[pl.ds(i * L, L)]
                        gidx = pk & 0xFFFF
                        gidx = jnp.minimum(jnp.maximum(gidx, 0), N_tab - 1)
                        g = i * L - (i // 8) * 128
                        ib2[i // 8, pl.ds(g, L)] = gidx
                        return 0

                    lax.fori_loop(0, K2 // L, ivec, 0)
                    for q in range(K2 // 128):
                        pltpu.async_copy(tab.at[ib2.at[q]],
                                         rows.at[pl.ds(q * 128, 128)],
                                         sem).wait()
                    m2 = jnp.minimum(K2, n - j * K2)

                    def aedge(k, _):
                        loc = pkb[pl.ds(k, L)][0] >> 16
                        for cc in range(C // L):
                            s = pl.ds(cc * L, L)
                            v = rows[k, s]
                            acc_s[loc, s] = acc_s[loc, s] + v
                            acc_m[loc, s] = jnp.maximum(acc_m[loc, s], v)
                        return 0

                    lax.fori_loop(0, m2, aedge, 0)
                    return 0

                lax.fori_loop(0, (n + K2 - 1) // K2, asub, 0)
                return 0

            lax.fori_loop(0, NW, region, 0)

            @pl.when(b < NW - 1)
            def _():
                pltpu.sync_copy(acc_s,
                                sum_hbm.at[pl.ds(lo, R), pl.ds(p * C, C)])
                pltpu.sync_copy(acc_m,
                                max_hbm.at[pl.ds(lo, R), pl.ds(p * C, C)])

            @pl.when(b == NW - 1)
            def _():
                pltpu.sync_copy(acc_s.at[pl.ds(0, R_last)],
                                sum_hbm.at[pl.ds(lo, R_last), pl.ds(p * C, C)])
                pltpu.sync_copy(acc_m.at[pl.ds(0, R_last)],
                                max_hbm.at[pl.ds(lo, R_last), pl.ds(p * C, C)])

    return pl.kernel(
        body,
        out_type=(
            jax.ShapeDtypeStruct((N_own, C * P), jnp.float32),
            jax.ShapeDtypeStruct((N_own, C * P), jnp.float32),
        ),
        mesh=_SC_MESH(),
        compiler_params=_UNTILE,
        scratch_types=[
            pltpu.VMEM((NW * CPAD,), jnp.int32),
            pltpu.VMEM((K2 + L,), jnp.int32),
            pltpu.VMEM((K2 // 128, 128), jnp.int32),
            pltpu.VMEM((K2, C), jnp.float32),
            pltpu.VMEM((R, C), jnp.float32),
            pltpu.VMEM((R, C), jnp.float32),
            pltpu.SemaphoreType.DMA,
        ],
    )


def _make_wscatter_kernel(C, P):
    RT = N_CELL // NS
    ZR = RT // 5

    def body(dst_hbm, src_hbm, ew_hbm, *rest):
        tabs = rest[:P]
        (out_hbm, db, sb, eb, s2, rows, zbuf, acc_sh, sem) = rest[P:]
        cid = lax.axis_index("c")
        sid = lax.axis_index("s")
        ebase = _wid() * ET

        for p in range(P):
            def zv(i, _):
                for cc in range(C // L):
                    zbuf[i, pl.ds(cc * L, L)] = jnp.zeros((L,), jnp.float32)
                return 0

            lax.fori_loop(0, ZR, zv, 0)

            def zcp(q, _):
                pltpu.sync_copy(zbuf, acc_sh.at[pl.ds(sid * RT + q * ZR, ZR)])
                return 0

            lax.fori_loop(0, 5, zcp, 0)
            plsc.subcore_barrier()

            def chunk(ch, _):
                base = ebase + ch * KC
                pltpu.sync_copy(dst_hbm.at[pl.ds(base, KC)], db)
                pltpu.sync_copy(src_hbm.at[pl.ds(base, KC)], sb)
                pltpu.sync_copy(ew_hbm.at[pl.ds(base, KC)],
                                eb.at[pl.ds(0, KC)])

                def do_group(g, _):
                    def ld(i, _):
                        dv = db[pl.ds(g * 128 + i * L, L)]
                        s2[0, pl.ds(i * L, L)] = jnp.minimum(dv, N_NET - 1)
                        svv = sb[pl.ds(g * 128 + i * L, L)]
                        s2[1, pl.ds(i * L, L)] = jnp.minimum(svv, N_CELL - 1)
                        return 0

                    lax.fori_loop(0, 128 // L, ld, 0)
                    pltpu.async_copy(tabs[p].at[s2.at[0]], rows, sem).wait()

                    def mulv(k, _):
                        w = eb[pl.ds(g * 128 + k, L)][0]
                        for cc in range(C // L):
                            s = pl.ds(cc * L, L)
                            rows[k, s] = rows[k, s] * w
                        return 0

                    lax.fori_loop(0, 128, mulv, 0)
                    pltpu.sync_copy(rows, acc_sh.at[s2.at[1]], add=True)
                    return 0

                lax.fori_loop(0, NGRP, do_group, 0)
                return 0

            lax.fori_loop(0, NCHP, chunk, 0)
            plsc.subcore_barrier()

            def wout(q, _):
                ro = sid * RT + q * ZR
                pltpu.sync_copy(acc_sh.at[pl.ds(ro, ZR)], zbuf)
                pltpu.sync_copy(
                    zbuf,
                    out_hbm.at[pl.ds(cid * N_CELL + ro, ZR), pl.ds(p * C, C)])
                return 0

            lax.fori_loop(0, 5, wout, 0)
            plsc.subcore_barrier()

    return pl.kernel(
        body,
        out_type=jax.ShapeDtypeStruct((NC * N_CELL, C * P), jnp.float32),
        mesh=_SC_MESH(),
        compiler_params=_UNTILE,
        scratch_types=[
            pltpu.VMEM((KC,), jnp.int32),
            pltpu.VMEM((KC,), jnp.int32),
            pltpu.VMEM((KC + L,), jnp.float32),
            pltpu.VMEM((2, 128), jnp.int32),
            pltpu.VMEM((128, C), jnp.float32),
            pltpu.VMEM((RT // 5, C), jnp.float32),
            pltpu.VMEM_SHARED((N_CELL, C), jnp.float32),
            pltpu.SemaphoreType.DMA,
        ],
    )


def _make_gather_kernel(H, Kc):
    ETg = EP_PAD // NW
    NCH = ETg // Kc

    def body(hc_hbm, hn_hbm, fa_hbm, so_hbm, gf_hbm, fsn_hbm, gfn_hbm,
             cs0_hbm, cs1_hbm,
             g_fa, g_so, g_gf, g_fsn, g_gfn, bnd_hbm,
             ixb, i2, rows, v0, v1, v2, v3, bb, sem):
        ebase = _wid() * ETg

        def chunk(ch, _):
            base = ebase + ch * Kc
            for idx_hbm, tab, out in ((fa_hbm, hc_hbm, g_fa),
                                      (so_hbm, hc_hbm, g_so),
                                      (gf_hbm, hc_hbm, g_gf),
                                      (fsn_hbm, hn_hbm, g_fsn),
                                      (gfn_hbm, hn_hbm, g_gfn)):
                pltpu.sync_copy(idx_hbm.at[pl.ds(base, Kc)], ixb)

                def mk(i, _):
                    g = i * L - (i // 8) * 128
                    i2[i // 8, pl.ds(g, L)] = ixb[pl.ds(i * L, L)]
                    return 0

                lax.fori_loop(0, Kc // L, mk, 0)

                for q in range(Kc // 128):
                    pltpu.async_copy(tab.at[i2.at[q]],
                                     rows.at[pl.ds(q * 128, 128)],
                                     sem).wait()
                pltpu.sync_copy(rows, out.at[pl.ds(base, Kc)])

            for idx_hbm, o0, o1 in ((fa_hbm, v0, v1), (so_hbm, v2, v3)):
                pltpu.sync_copy(idx_hbm.at[pl.ds(base, Kc)], ixb)

                def mk2(i, _):
                    g = i * L - (i // 8) * 128
                    i2[i // 8, pl.ds(g, L)] = ixb[pl.ds(i * L, L)]
                    return 0

                lax.fori_loop(0, Kc // L, mk2, 0)
                for q in range(Kc // 128):
                    pltpu.async_copy(cs0_hbm.at[i2.at[q]],
                                     o0.at[pl.ds(q * 128, 128)], sem).wait()
                    pltpu.async_copy(cs1_hbm.at[i2.at[q]],
                                     o1.at[pl.ds(q * 128, 128)], sem).wait()

            def bvec(i, _):
                s = pl.ds(i * L, L)
                bb[s] = jnp.minimum(v0[s] + v2[s], v1[s] + v3[s]) * 0.5
                return 0

            lax.fori_loop(0, Kc // L, bvec, 0)
            pltpu.sync_copy(bb, bnd_hbm.at[pl.ds(base, Kc)])
            return 0

        lax.fori_loop(0, NCH, chunk, 0)

    return pl.kernel(
        body,
        out_type=tuple(
            [jax.ShapeDtypeStruct((EP_PAD, H), jnp.float32)] * 5
            + [jax.ShapeDtypeStruct((EP_PAD,), jnp.float32)]),
        mesh=_SC_MESH(),
        scratch_types=[
            pltpu.VMEM((Kc,), jnp.int32),
            pltpu.VMEM((Kc // 128, 128), jnp.int32),
            pltpu.VMEM((Kc, H), jnp.float32),
            pltpu.VMEM((Kc,), jnp.float32),
            pltpu.VMEM((Kc,), jnp.float32),
            pltpu.VMEM((Kc,), jnp.float32),
            pltpu.VMEM((Kc,), jnp.float32),
            pltpu.VMEM((Kc,), jnp.float32),
            pltpu.SemaphoreType.DMA,
        ],
    )


def _enc_body(raw, ssum, smax, c0, c1, w1, w2, w3, b, out):
    c = c0[...] + c1[...]
    rcp = 1.0 / jnp.maximum(c, 1.0)
    mean = ssum[...] * rcp
    mx = jnp.where(c > 0.0, smax[...], 0.0)
    acc = jnp.dot(raw[...], w1[...], preferred_element_type=jnp.float32)
    acc += jnp.dot(mean, w2[...], preferred_element_type=jnp.float32)
    acc += jnp.dot(mx, w3[...], preferred_element_type=jnp.float32)
    out[...] = jnp.tanh(acc + b[...])


def _encoder(raw, ssum, smax, cnt2, W, b, blk):
    n, r = raw.shape
    h = W.shape[1]
    w1, w2, w3 = W[:r], W[r:2 * r], W[2 * r:]
    return pl.pallas_call(
        _enc_body,
        grid=(n // blk,),
        in_specs=[
            pl.BlockSpec((blk, r), lambda i: (i, 0)),
            pl.BlockSpec((blk, r), lambda i: (i, 0)),
            pl.BlockSpec((blk, r), lambda i: (i, 0)),
            pl.BlockSpec((blk, 1), lambda i: (i, 0)),
            pl.BlockSpec((blk, 1), lambda i: (i, 0)),
            pl.BlockSpec((r, h), lambda i: (0, 0)),
            pl.BlockSpec((r, h), lambda i: (0, 0)),
            pl.BlockSpec((r, h), lambda i: (0, 0)),
            pl.BlockSpec((1, h), lambda i: (0, 0)),
        ],
        out_specs=pl.BlockSpec((blk, h), lambda i: (i, 0)),
        out_shape=jax.ShapeDtypeStruct((n, h), jnp.float32),
    )(raw, ssum, smax, cnt2[0].reshape(n, 1), cnt2[1].reshape(n, 1),
      w1, w2, w3, b.reshape(1, h))


def _pin_body(praw, wp, bp, we, be, out):
    hp = jnp.tanh(jnp.dot(praw[...], wp[...],
                          preferred_element_type=jnp.float32) + bp[...])
    out[...] = jnp.tanh(jnp.dot(hp, we[...],
                                preferred_element_type=jnp.float32) + be[...])


def _pin_mlp(pin_raw, W_pin, b_pin, W_ew, b_ew, blk):
    n, r = pin_raw.shape
    hp = W_pin.shape[1]
    return pl.pallas_call(
        _pin_body,
        grid=(n // blk,),
        in_specs=[
            pl.BlockSpec((blk, r), lambda i: (i, 0)),
            pl.BlockSpec((r, hp), lambda i: (0, 0)),
            pl.BlockSpec((1, hp), lambda i: (0, 0)),
            pl.BlockSpec((hp, 1), lambda i: (0, 0)),
            pl.BlockSpec((1, 1), lambda i: (0, 0)),
        ],
        out_specs=pl.BlockSpec((blk, 1), lambda i: (i, 0)),
        out_shape=jax.ShapeDtypeStruct((n, 1), jnp.float32),
    )(pin_raw, W_pin, b_pin.reshape(1, hp), W_ew, b_ew.reshape(1, 1))


def _upd_body(hc0, np0, np1, c0, c1, ws, wn, b, out):
    rcp = 1.0 / jnp.maximum(c0[...] + c1[...], 1.0)
    neigh = (np0[...] + np1[...]) * rcp
    acc = jnp.dot(hc0[...], ws[...], preferred_element_type=jnp.float32)
    acc += jnp.dot(neigh, wn[...], preferred_element_type=jnp.float32)
    out[...] = acc + b[...]


def _sage_update(hc0, neigh_part, cnt2, W_self, W_neigh, bias, blk):
    n, h = hc0.shape
    return pl.pallas_call(
        _upd_body,
        grid=(n // blk,),
        in_specs=[
            pl.BlockSpec((blk, h), lambda i: (i, 0)),
            pl.BlockSpec((blk, h), lambda i: (i, 0)),
            pl.BlockSpec((blk, h), lambda i: (i, 0)),
            pl.BlockSpec((blk, 1), lambda i: (i, 0)),
            pl.BlockSpec((blk, 1), lambda i: (i, 0)),
            pl.BlockSpec((h, h), lambda i: (0, 0)),
            pl.BlockSpec((h, h), lambda i: (0, 0)),
            pl.BlockSpec((1, h), lambda i: (0, 0)),
        ],
        out_specs=pl.BlockSpec((blk, h), lambda i: (i, 0)),
        out_shape=jax.ShapeDtypeStruct((n, h), jnp.float32),
    )(hc0, neigh_part[0], neigh_part[1], cnt2[0].reshape(n, 1),
      cnt2[1].reshape(n, 1), W_self, W_neigh, bias.reshape(1, h))


def _readout_body(fa, so, gf, fsn, gfn, bnd,
                  a1, a2, a3, bd1, wd2, bd2, wd3, bd3,
                  f1, f2, f3, f4, f5, bf1, wf2, bf2, wf3, bf3,
                  dis_o, defl_o):
    x = jnp.dot(fa[...], a1[...], preferred_element_type=jnp.float32)
    x += jnp.dot(so[...], a2[...], preferred_element_type=jnp.float32)
    x += jnp.dot(fsn[...], a3[...], preferred_element_type=jnp.float32)
    h1 = jax.nn.relu(x + bd1[...])
    h1 = jax.nn.relu(jnp.dot(h1, wd2[...],
                             preferred_element_type=jnp.float32) + bd2[...])
    dis = jnp.dot(h1, wd3[...], preferred_element_type=jnp.float32) + bd3[...]

    y = jnp.dot(gf[...], f1[...], preferred_element_type=jnp.float32)
    y += jnp.dot(fa[...], f2[...], preferred_element_type=jnp.float32)
    y += jnp.dot(so[...], f3[...], preferred_element_type=jnp.float32)
    y += jnp.dot(gfn[...], f4[...], preferred_element_type=jnp.float32)
    y += jnp.dot(fsn[...], f5[...], preferred_element_type=jnp.float32)
    h2 = jax.nn.relu(y + bf1[...])
    h2 = jax.nn.relu(jnp.dot(h2, wf2[...],
                             preferred_element_type=jnp.float32) + bf2[...])
    defl = jnp.dot(h2, wf3[...], preferred_element_type=jnp.float32) + bf3[...]

    dis_o[...] = jnp.exp(-2.0 + 15.0 * jnp.tanh(dis)) + bnd[...]
    defl_o[...] = jnp.tanh(defl) * (2.0 * jnp.pi)


def _readout(g_fa, g_so, g_gf, g_fsn, g_gfn, bnd,
             Wd1, bd1, Wd2, bd2, Wd3, bd3,
             Wf1, bf1, Wf2, bf2, Wf3, bf3, blk):
    n, h = g_fa.shape
    d1 = Wd1.shape[1]
    d2 = Wd2.shape[1]
    e1 = Wf1.shape[1]
    e2 = Wf2.shape[1]
    a1, a2, a3 = Wd1[:h], Wd1[h:2 * h], Wd1[2 * h:]
    f1, f2, f3, f4, f5 = (Wf1[:h], Wf1[h:2 * h], Wf1[2 * h:3 * h],
                          Wf1[3 * h:4 * h], Wf1[4 * h:])
    row = lambda i: (i, 0)
    fix = lambda i: (0, 0)
    return pl.pallas_call(
        _readout_body,
        grid=(n // blk,),
        in_specs=[
            pl.BlockSpec((blk, h), row), pl.BlockSpec((blk, h), row),
            pl.BlockSpec((blk, h), row), pl.BlockSpec((blk, h), row),
            pl.BlockSpec((blk, h), row), pl.BlockSpec((blk, 1), row),
            pl.BlockSpec((h, d1), fix), pl.BlockSpec((h, d1), fix),
            pl.BlockSpec((h, d1), fix), pl.BlockSpec((1, d1), fix),
            pl.BlockSpec((d1, d2), fix), pl.BlockSpec((1, d2), fix),
            pl.BlockSpec((d2, 1), fix), pl.BlockSpec((1, 1), fix),
            pl.BlockSpec((h, e1), fix), pl.BlockSpec((h, e1), fix),
            pl.BlockSpec((h, e1), fix), pl.BlockSpec((h, e1), fix),
            pl.BlockSpec((h, e1), fix), pl.BlockSpec((1, e1), fix),
            pl.BlockSpec((e1, e2), fix), pl.BlockSpec((1, e2), fix),
            pl.BlockSpec((e2, 1), fix), pl.BlockSpec((1, 1), fix),
        ],
        out_specs=(pl.BlockSpec((blk, 1), row), pl.BlockSpec((blk, 1), row)),
        out_shape=(jax.ShapeDtypeStruct((n, 1), jnp.float32),
                   jax.ShapeDtypeStruct((n, 1), jnp.float32)),
    )(g_fa, g_so, g_gf, g_fsn, g_gfn, bnd.reshape(n, 1),
      a1, a2, a3, bd1.reshape(1, d1), Wd2, bd2.reshape(1, d2),
      Wd3, bd3.reshape(1, 1),
      f1, f2, f3, f4, f5, bf1.reshape(1, e1), Wf2, bf2.reshape(1, e2),
      Wf3, bf3.reshape(1, 1))


_degrees = _make_degree_kernel()
_partition = _make_partition_kernel()
_stats_net = _make_stats_kernel(N_NET, N_CELL, R=R_NET, C=64, P=1)
_stats_cell = _make_stats_kernel(N_CELL, N_NET, R=R_CELL, C=32, P=2)
_wscatter = _make_wscatter_kernel(C=32, P=4)
_gather_ro = _make_gather_kernel(128, Kc=640)


def kernel(cell_raw, net_raw, pin_raw, cell_size,
           pin_src, pin_dst, fathers, sons, grandfathers, fs_nets, gf_nets,
           W_cell, b_cell, W_net, b_net, W_pin, b_pin, W_ew, b_ew,
           W_self, b_self, W_neigh, b_neigh,
           Wd1, bd1, Wd2, bd2, Wd3, bd3,
           Wf1, bf1, Wf2, bf2, Wf3, bf3):
    padn = E_PAD - E_PIN
    srcp = jnp.pad(pin_src.astype(jnp.int32), (0, padn),
                   constant_values=PAD_SRC)
    dstp = jnp.pad(pin_dst.astype(jnp.int32), (0, padn),
                   constant_values=PAD_DST)

    net_lo = jnp.asarray(net_raw[:, :32])
    net_hi = jnp.asarray(net_raw[:, 32:])

    deg_cell2, deg_net2 = _degrees(srcp, dstp)
    deg_cell2 = deg_cell2.reshape(NC, N_CELL)
    deg_net2 = deg_net2.reshape(NC, N_NET)

    pkc, cntc, pkn, cntn = _partition(srcp, dstp)

    net_sum, net_max = _stats_net(pkn, cntn, cell_raw)
    cell_sum, cell_max = _stats_cell(pkc, cntc, net_lo, net_hi)

    hc0 = _encoder(cell_raw, cell_sum, cell_max, deg_cell2,
                   W_cell, b_cell, blk=1000)
    hn = _encoder(net_raw, net_sum, net_max, deg_net2, W_net, b_net, blk=1000)

    ew = _pin_mlp(pin_raw, W_pin, b_pin, W_ew, b_ew, blk=4000)[:, 0]
    ewp = jnp.pad(ew, (0, padn))

    hn_parts = [jnp.asarray(hn[:, i * 32:(i + 1) * 32]) for i in range(4)]
    neigh_part = _wscatter(dstp, srcp, ewp, *hn_parts).reshape(
        NC, N_CELL, 128)

    hc = _sage_update(hc0, neigh_part, deg_cell2, W_self, W_neigh,
                      b_self + b_neigh, blk=1000)

    pad = EP_PAD - E_PT
    fa_p = jnp.pad(fathers.astype(jnp.int32), (0, pad))
    so_p = jnp.pad(sons.astype(jnp.int32), (0, pad))
    gf_p = jnp.pad(grandfathers.astype(jnp.int32), (0, pad))
    fsn_p = jnp.pad(fs_nets.astype(jnp.int32), (0, pad))
    gfn_p = jnp.pad(gf_nets.astype(jnp.int32), (0, pad))
    cs0 = jnp.asarray(cell_size[:, 0])
    cs1 = jnp.asarray(cell_size[:, 1])

    g_fa, g_so, g_gf, g_fsn, g_gfn, bnd = _gather_ro(
        hc, hn, fa_p, so_p, gf_p, fsn_p, gfn_p, cs0, cs1)

    dis, defl = _readout(g_fa, g_so, g_gf, g_fsn, g_gfn, bnd,
                         Wd1, bd1, Wd2, bd2, Wd3, bd3,
                         Wf1, bf1, Wf2, bf2, Wf3, bf3, blk=512)

    return (dis[:E_PT, 0], defl[:E_PT, 0])
